# Initial kernel scaffold; baseline (speedup 1.0000x reference)
import jax
import jax.numpy as jnp
from jax.experimental import pallas as pl


def kernel(drug_ids, disease_ids_i, disease_ids_j, x, edge_index, W1, b1, W2, b2):
    raise NotImplementedError("write your pallas kernel here")


# speedup vs baseline: 9.3532x; 9.3532x over previous
import functools

import jax
import jax.numpy as jnp
from jax import lax
from jax.experimental import pallas as pl
from jax.experimental.pallas import tpu as pltpu
from jax.experimental.pallas import tpu_sc as plsc

N_NODES = 10000
IN_CH = 128
HID_CH = 128
FACTOR = 64
N_EDGES = 320000
BATCH = 4096

NC = 2
NS = 16
NW = NC * NS
LANES = 16

CH = 128
NCHUNK = 80
EPT = NCHUNK * CH
EPAD = NW * EPT
NPAD = 10112
RPT = NPAD // NS
BPT = BATCH // NW

def _mesh():
    return plsc.VectorSubcoreMesh(
        core_axis_name="c", subcore_axis_name="s", num_cores=NC, num_subcores=NS
    )


@functools.partial(
    pl.kernel,
    out_type=jax.ShapeDtypeStruct((NW * NPAD,), jnp.float32),
    mesh=_mesh(),
    scratch_types=[
        pltpu.VMEM((NCHUNK, CH), jnp.int32),
        pltpu.VMEM((NPAD,), jnp.float32),
    ],
    compiler_params=pltpu.CompilerParams(needs_layout_passes=False),
)
def _deg_kernel(dsts_hbm, out_hbm, dst_v, hist):
    c = lax.axis_index("c")
    s = lax.axis_index("s")
    wid = s * NC + c
    pltpu.sync_copy(dsts_hbm.at[wid], dst_v)

    def zbody(k, carry):
        hist[pl.ds(k * LANES, LANES)] = jnp.zeros((LANES,), jnp.float32)
        return carry

    lax.fori_loop(0, NPAD // LANES, zbody, 0)
    ones = jnp.ones((LANES,), jnp.float32)

    def body(j, carry):
        def inner(k, carry2):
            idx = dst_v[j, pl.ds(k * LANES, LANES)]
            plsc.addupdate_scatter(hist, [idx], ones)
            return carry2

        return lax.fori_loop(0, CH // LANES, inner, carry)

    lax.fori_loop(0, NCHUNK, body, 0)
    pltpu.sync_copy(hist, out_hbm.at[pl.ds(wid * NPAD, NPAD)])


def _make_scatter(width):
    @functools.partial(
        pl.kernel,
        out_type=jax.ShapeDtypeStruct((NC, NPAD, width), jnp.float32),
        mesh=_mesh(),
        scratch_types=[
            pltpu.VMEM((NCHUNK, CH), jnp.int32),
            pltpu.VMEM((NCHUNK, CH), jnp.int32),
            pltpu.VMEM((CH, width), jnp.float32),
            pltpu.VMEM_SHARED((NPAD, width), jnp.float32),
            pltpu.SemaphoreType.DMA,
        ],
    )
    def _scatter_kernel(y_hbm, srcs_hbm, dsts_hbm, zeros_hbm, out_hbm,
                        src_v, dst_v, rowbuf, acc, sem):
        c = lax.axis_index("c")
        s = lax.axis_index("s")
        wid = s * NC + c
        pltpu.sync_copy(srcs_hbm.at[wid], src_v)
        pltpu.sync_copy(dsts_hbm.at[wid], dst_v)
        r0 = s * RPT
        pltpu.sync_copy(zeros_hbm.at[pl.ds(r0, RPT)], acc.at[pl.ds(r0, RPT)])
        plsc.subcore_barrier()

        def body(j, carry):
            pltpu.async_copy(y_hbm.at[src_v.at[j]], rowbuf, sem).wait()
            pltpu.sync_copy(rowbuf, acc.at[dst_v.at[j]], add=True)
            return carry

        lax.fori_loop(0, NCHUNK, body, 0)
        plsc.subcore_barrier()
        pltpu.sync_copy(acc.at[pl.ds(r0, RPT)], out_hbm.at[c, pl.ds(r0, RPT)])

    return _scatter_kernel


_scatter128 = _make_scatter(HID_CH)


@functools.partial(
    pl.kernel,
    out_type=jax.ShapeDtypeStruct((3, BATCH, HID_CH), jnp.float32),
    mesh=_mesh(),
    scratch_types=[
        pltpu.VMEM((BPT,), jnp.int32),
        pltpu.VMEM((BPT, HID_CH), jnp.float32),
        pltpu.SemaphoreType.DMA,
    ],
)
def _gather_kernel(emb_hbm, ids0_hbm, ids1_hbm, ids2_hbm, out_hbm,
                   idx_v, rowbuf, sem):
    c = lax.axis_index("c")
    s = lax.axis_index("s")
    wid = s * NC + c
    base = wid * BPT
    for t, ids_hbm in enumerate((ids0_hbm, ids1_hbm, ids2_hbm)):
        pltpu.sync_copy(ids_hbm.at[pl.ds(base, BPT)], idx_v)
        pltpu.async_copy(emb_hbm.at[idx_v], rowbuf, sem).wait()
        pltpu.sync_copy(rowbuf, out_hbm.at[t, pl.ds(base, BPT)])


def _dinv(deg_ref):
    deg = jnp.sum(deg_ref[...], axis=0) + 1.0
    return lax.rsqrt(deg)[:, None]


def _b_body(x_ref, w_ref, deg_ref, y_ref):
    h = jnp.dot(x_ref[...], w_ref[...], preferred_element_type=jnp.float32)
    y_ref[...] = h * _dinv(deg_ref)


_tc_b = pl.pallas_call(
    _b_body, out_shape=jax.ShapeDtypeStruct((NPAD, HID_CH), jnp.float32)
)


def _d_body(p_ref, y1_ref, deg_ref, b1_ref, u_ref):
    dinv = _dinv(deg_ref)
    z = p_ref[0] + p_ref[1] + y1_ref[...]
    h1 = jnp.maximum(z * dinv + b1_ref[...], 0.0)
    u_ref[...] = h1 * dinv


_tc_d = pl.pallas_call(
    _d_body, out_shape=jax.ShapeDtypeStruct((NPAD, HID_CH), jnp.float32)
)


def _f_body(q_ref, u_ref, deg_ref, w2_ref, b2_ref, emb_ref):
    z2 = (q_ref[0] + q_ref[1] + u_ref[...]) * _dinv(deg_ref)
    e = jnp.dot(z2, w2_ref[...], preferred_element_type=jnp.float32) + b2_ref[...]
    emb_ref[...] = jnp.concatenate(
        [e, jnp.zeros((NPAD, HID_CH - FACTOR), jnp.float32)], axis=1
    )


_tc_f = pl.pallas_call(
    _f_body, out_shape=jax.ShapeDtypeStruct((NPAD, HID_CH), jnp.float32)
)


def _h_body(rows_ref, out_ref):
    a = rows_ref[0]
    pi = jnp.sum(a * rows_ref[1], axis=-1, keepdims=True)
    pj = jnp.sum(a * rows_ref[2], axis=-1, keepdims=True)
    out_ref[...] = jnp.concatenate([pi, pj], axis=1)


_tc_h = pl.pallas_call(
    _h_body, out_shape=jax.ShapeDtypeStruct((BATCH, 2), jnp.float32)
)


def kernel(drug_ids, disease_ids_i, disease_ids_j, x, edge_index, W1, b1, W2, b2):
    src = edge_index[0].astype(jnp.int32)
    dst = edge_index[1].astype(jnp.int32)
    pad = jnp.full((EPAD - N_EDGES,), N_NODES, jnp.int32)
    srcs = jnp.concatenate([src, pad]).reshape(NW, NCHUNK, CH)
    dsts = jnp.concatenate([dst, pad]).reshape(NW, NCHUNK, CH)

    deg2 = _deg_kernel(dsts).reshape(NW, NPAD)

    x_p = jnp.concatenate([x, jnp.zeros((NPAD - N_NODES, IN_CH), x.dtype)])
    y1 = _tc_b(x_p, W1, deg2)

    zeros128 = jnp.zeros((NPAD, HID_CH), jnp.float32)
    p = _scatter128(y1, srcs, dsts, zeros128)
    u = _tc_d(p, y1, deg2, b1)

    q = _scatter128(u, srcs, dsts, zeros128)
    emb_p = _tc_f(q, u, deg2, W2, b2)

    rows = _gather_kernel(
        emb_p,
        drug_ids.astype(jnp.int32),
        disease_ids_i.astype(jnp.int32),
        disease_ids_j.astype(jnp.int32),
    )
    preds = _tc_h(rows)
    return (preds[:, 0], preds[:, 1], emb_p[:N_NODES, :FACTOR])


# speedup vs baseline: 32.4768x; 3.4723x over previous
import functools

import jax
import jax.numpy as jnp
from jax import lax
from jax.experimental import pallas as pl
from jax.experimental.pallas import tpu as pltpu
from jax.experimental.pallas import tpu_sc as plsc

N_NODES = 10000
IN_CH = 128
HID_CH = 128
FACTOR = 64
N_EDGES = 320000
BATCH = 4096

NC = 2
NS = 16
NW = NC * NS
LANES = 16

CH = 64
NCHUNK = 160
NPHASE = 4
NCHP = NCHUNK // NPHASE
EPT = NCHUNK * CH
EPAD = NW * EPT
NPAD = 10112
RPT = NPAD // NS
BPT = BATCH // NW

def _mesh():
    return plsc.VectorSubcoreMesh(
        core_axis_name="c", subcore_axis_name="s", num_cores=NC, num_subcores=NS
    )


@functools.partial(
    pl.kernel,
    out_type=jax.ShapeDtypeStruct((NW * NPAD,), jnp.float32),
    mesh=_mesh(),
    scratch_types=[
        pltpu.VMEM((NCHUNK, CH), jnp.int32),
        pltpu.VMEM((NPAD,), jnp.float32),
    ],
    compiler_params=pltpu.CompilerParams(needs_layout_passes=False),
)
def _deg_kernel(dsts_hbm, out_hbm, dst_v, hist):
    c = lax.axis_index("c")
    s = lax.axis_index("s")
    wid = s * NC + c
    pltpu.sync_copy(dsts_hbm.at[wid], dst_v)

    def zbody(k, carry):
        hist[pl.ds(k * LANES, LANES)] = jnp.zeros((LANES,), jnp.float32)
        return carry

    lax.fori_loop(0, NPAD // LANES, zbody, 0)
    ones = jnp.ones((LANES,), jnp.float32)

    def body(j, carry):
        def inner(k, carry2):
            idx = dst_v[j, pl.ds(k * LANES, LANES)]
            plsc.addupdate_scatter(hist, [idx], ones)
            return carry2

        return lax.fori_loop(0, CH // LANES, inner, carry)

    lax.fori_loop(0, NCHUNK, body, 0)
    pltpu.sync_copy(hist, out_hbm.at[pl.ds(wid * NPAD, NPAD)])


NBUF = 4


def _make_scatter(width):
    @functools.partial(
        pl.kernel,
        out_type=jax.ShapeDtypeStruct((NC, NPAD, width), jnp.float32),
        mesh=_mesh(),
        scratch_types=(
            [
                pltpu.VMEM((NCHP, CH), jnp.int32),
                pltpu.VMEM((NCHP, CH), jnp.int32),
                pltpu.VMEM_SHARED((NPAD, width), jnp.float32),
            ]
            + [pltpu.VMEM((CH, width), jnp.float32)] * NBUF
            + [pltpu.SemaphoreType.DMA] * NBUF
        ),
    )
    def _scatter_kernel(y_hbm, srcs_hbm, dsts_hbm, zeros_hbm, out_hbm,
                        src_v, dst_v, acc, *bufs_sems):
        rowbufs = bufs_sems[:NBUF]
        sems = bufs_sems[NBUF:]
        c = lax.axis_index("c")
        s = lax.axis_index("s")
        wid = s * NC + c
        r0 = s * RPT
        pltpu.sync_copy(zeros_hbm.at[pl.ds(r0, RPT)], acc.at[pl.ds(r0, RPT)])
        plsc.subcore_barrier()

        for phase in range(NPHASE):
            pltpu.sync_copy(srcs_hbm.at[wid, pl.ds(phase * NCHP, NCHP)], src_v)
            pltpu.sync_copy(dsts_hbm.at[wid, pl.ds(phase * NCHP, NCHP)], dst_v)
            for b in range(NBUF):
                pltpu.async_copy(y_hbm.at[src_v.at[b]], rowbufs[b], sems[b])

            def body(g, carry):
                for b in range(NBUF):
                    j = g * NBUF + b
                    pltpu.make_async_copy(y_hbm.at[src_v.at[j]], rowbufs[b],
                                          sems[b]).wait()
                    pltpu.sync_copy(rowbufs[b], acc.at[dst_v.at[j]], add=True)
                    jn = j + NBUF

                    @pl.when(jn < NCHP)
                    def _():
                        pltpu.async_copy(y_hbm.at[src_v.at[jn]], rowbufs[b],
                                         sems[b])

                return carry

            lax.fori_loop(0, NCHP // NBUF, body, 0)

        plsc.subcore_barrier()
        pltpu.sync_copy(acc.at[pl.ds(r0, RPT)], out_hbm.at[c, pl.ds(r0, RPT)])

    return _scatter_kernel


_scatter128 = _make_scatter(HID_CH)


@functools.partial(
    pl.kernel,
    out_type=jax.ShapeDtypeStruct((3, BATCH, HID_CH), jnp.float32),
    mesh=_mesh(),
    scratch_types=[
        pltpu.VMEM((BPT,), jnp.int32),
        pltpu.VMEM((BPT, HID_CH), jnp.float32),
        pltpu.SemaphoreType.DMA,
    ],
)
def _gather_kernel(emb_hbm, ids0_hbm, ids1_hbm, ids2_hbm, out_hbm,
                   idx_v, rowbuf, sem):
    c = lax.axis_index("c")
    s = lax.axis_index("s")
    wid = s * NC + c
    base = wid * BPT
    for t, ids_hbm in enumerate((ids0_hbm, ids1_hbm, ids2_hbm)):
        pltpu.sync_copy(ids_hbm.at[pl.ds(base, BPT)], idx_v)
        pltpu.async_copy(emb_hbm.at[idx_v], rowbuf, sem).wait()
        pltpu.sync_copy(rowbuf, out_hbm.at[t, pl.ds(base, BPT)])


def _dinv(deg_ref):
    deg = jnp.sum(deg_ref[...], axis=0) + 1.0
    return lax.rsqrt(deg)[:, None]


def _b_body(x_ref, w_ref, deg_ref, y_ref):
    h = jnp.dot(x_ref[...], w_ref[...], preferred_element_type=jnp.float32)
    y_ref[...] = h * _dinv(deg_ref)


_tc_b = pl.pallas_call(
    _b_body, out_shape=jax.ShapeDtypeStruct((NPAD, HID_CH), jnp.float32)
)


def _d_body(p_ref, y1_ref, deg_ref, b1_ref, u_ref):
    dinv = _dinv(deg_ref)
    z = p_ref[0] + p_ref[1] + y1_ref[...]
    h1 = jnp.maximum(z * dinv + b1_ref[...], 0.0)
    u_ref[...] = h1 * dinv


_tc_d = pl.pallas_call(
    _d_body, out_shape=jax.ShapeDtypeStruct((NPAD, HID_CH), jnp.float32)
)


def _f_body(q_ref, u_ref, deg_ref, w2_ref, b2_ref, emb_ref):
    z2 = (q_ref[0] + q_ref[1] + u_ref[...]) * _dinv(deg_ref)
    e = jnp.dot(z2, w2_ref[...], preferred_element_type=jnp.float32) + b2_ref[...]
    emb_ref[...] = jnp.concatenate(
        [e, jnp.zeros((NPAD, HID_CH - FACTOR), jnp.float32)], axis=1
    )


_tc_f = pl.pallas_call(
    _f_body, out_shape=jax.ShapeDtypeStruct((NPAD, HID_CH), jnp.float32)
)


def _h_body(rows_ref, out_ref):
    a = rows_ref[0]
    pi = jnp.sum(a * rows_ref[1], axis=-1, keepdims=True)
    pj = jnp.sum(a * rows_ref[2], axis=-1, keepdims=True)
    out_ref[...] = jnp.concatenate([pi, pj], axis=1)


_tc_h = pl.pallas_call(
    _h_body, out_shape=jax.ShapeDtypeStruct((BATCH, 2), jnp.float32)
)


def kernel(drug_ids, disease_ids_i, disease_ids_j, x, edge_index, W1, b1, W2, b2):
    src = edge_index[0].astype(jnp.int32)
    dst = edge_index[1].astype(jnp.int32)
    pad = N_NODES + jnp.arange(EPAD - N_EDGES, dtype=jnp.int32) % (NPAD - N_NODES)
    srcs = jnp.concatenate([src, pad]).reshape(NW, NCHUNK, CH)
    dsts = jnp.concatenate([dst, pad]).reshape(NW, NCHUNK, CH)

    deg2 = _deg_kernel(dsts).reshape(NW, NPAD)

    x_p = jnp.concatenate([x, jnp.zeros((NPAD - N_NODES, IN_CH), x.dtype)])
    y1 = _tc_b(x_p, W1, deg2)

    zeros128 = jnp.zeros((NPAD, HID_CH), jnp.float32)
    p = _scatter128(y1, srcs, dsts, zeros128)
    u = _tc_d(p, y1, deg2, b1)

    q = _scatter128(u, srcs, dsts, zeros128)
    emb_p = _tc_f(q, u, deg2, W2, b2)

    rows = _gather_kernel(
        emb_p,
        drug_ids.astype(jnp.int32),
        disease_ids_i.astype(jnp.int32),
        disease_ids_j.astype(jnp.int32),
    )
    preds = _tc_h(rows)
    return (preds[:, 0], preds[:, 1], emb_p[:N_NODES, :FACTOR])


# speedup vs baseline: 34.5270x; 1.0631x over previous
import functools

import jax
import jax.numpy as jnp
from jax import lax
from jax.experimental import pallas as pl
from jax.experimental.pallas import tpu as pltpu
from jax.experimental.pallas import tpu_sc as plsc

N_NODES = 10000
IN_CH = 128
HID_CH = 128
FACTOR = 64
N_EDGES = 320000
BATCH = 4096

NC = 2
NS = 16
NW = NC * NS
LANES = 16

CH = 64
NCHUNK = 160
NPHASE = 4
NCHP = NCHUNK // NPHASE
EPT = NCHUNK * CH
EPAD = NW * EPT
NPAD = 10112
RPT = NPAD // NS
BPT = BATCH // NW

def _mesh():
    return plsc.VectorSubcoreMesh(
        core_axis_name="c", subcore_axis_name="s", num_cores=NC, num_subcores=NS
    )


@functools.partial(
    pl.kernel,
    out_type=jax.ShapeDtypeStruct((NW * NPAD,), jnp.float32),
    mesh=_mesh(),
    scratch_types=[
        pltpu.VMEM((NCHUNK, CH), jnp.int32),
        pltpu.VMEM((NPAD,), jnp.float32),
    ],
    compiler_params=pltpu.CompilerParams(needs_layout_passes=False),
)
def _deg_kernel(dsts_hbm, out_hbm, dst_v, hist):
    c = lax.axis_index("c")
    s = lax.axis_index("s")
    wid = s * NC + c
    pltpu.sync_copy(dsts_hbm.at[wid], dst_v)

    def zbody(k, carry):
        hist[pl.ds(k * LANES, LANES)] = jnp.zeros((LANES,), jnp.float32)
        return carry

    lax.fori_loop(0, NPAD // LANES, zbody, 0)
    ones = jnp.ones((LANES,), jnp.float32)

    def body(j, carry):
        def inner(k, carry2):
            idx = dst_v[j, pl.ds(k * LANES, LANES)]
            plsc.addupdate_scatter(hist, [idx], ones)
            return carry2

        return lax.fori_loop(0, CH // LANES, inner, carry)

    lax.fori_loop(0, NCHUNK, body, 0)
    pltpu.sync_copy(hist, out_hbm.at[pl.ds(wid * NPAD, NPAD)])


NBUF = 4


def _make_scatter(width, tc_tiling=True):
    @functools.partial(
        pl.kernel,
        out_type=jax.ShapeDtypeStruct((NC, NPAD, width), jnp.float32),
        mesh=_mesh(),
        compiler_params=pltpu.CompilerParams(use_tc_tiling_on_sc=tc_tiling),
        scratch_types=(
            [
                pltpu.VMEM((NCHP, CH), jnp.int32),
                pltpu.VMEM((NCHP, CH), jnp.int32),
                pltpu.VMEM_SHARED((NPAD, width), jnp.float32),
            ]
            + [pltpu.VMEM((CH, width), jnp.float32)] * NBUF
            + [pltpu.SemaphoreType.DMA] * NBUF
        ),
    )
    def _scatter_kernel(y_hbm, srcs_hbm, dsts_hbm, zeros_hbm, out_hbm,
                        src_v, dst_v, acc, *bufs_sems):
        rowbufs = bufs_sems[:NBUF]
        sems = bufs_sems[NBUF:]
        c = lax.axis_index("c")
        s = lax.axis_index("s")
        wid = s * NC + c
        r0 = s * RPT
        pltpu.sync_copy(zeros_hbm.at[pl.ds(r0, RPT)], acc.at[pl.ds(r0, RPT)])
        plsc.subcore_barrier()

        for phase in range(NPHASE):
            pltpu.sync_copy(srcs_hbm.at[wid, pl.ds(phase * NCHP, NCHP)], src_v)
            pltpu.sync_copy(dsts_hbm.at[wid, pl.ds(phase * NCHP, NCHP)], dst_v)
            for b in range(NBUF):
                pltpu.async_copy(y_hbm.at[src_v.at[b]], rowbufs[b], sems[b])

            def body(g, carry):
                for b in range(NBUF):
                    j = g * NBUF + b
                    pltpu.make_async_copy(y_hbm.at[src_v.at[j]], rowbufs[b],
                                          sems[b]).wait()
                    pltpu.sync_copy(rowbufs[b], acc.at[dst_v.at[j]], add=True)
                    jn = j + NBUF

                    @pl.when(jn < NCHP)
                    def _():
                        pltpu.async_copy(y_hbm.at[src_v.at[jn]], rowbufs[b],
                                         sems[b])

                return carry

            lax.fori_loop(0, NCHP // NBUF, body, 0)

        plsc.subcore_barrier()
        pltpu.sync_copy(acc.at[pl.ds(r0, RPT)], out_hbm.at[c, pl.ds(r0, RPT)])

    return _scatter_kernel


_scatter128 = _make_scatter(HID_CH)
_scatter64 = _make_scatter(FACTOR, tc_tiling=False)


@functools.partial(
    pl.kernel,
    out_type=jax.ShapeDtypeStruct((3, BATCH, FACTOR), jnp.float32),
    mesh=_mesh(),
    compiler_params=pltpu.CompilerParams(use_tc_tiling_on_sc=False),
    scratch_types=[
        pltpu.VMEM((BPT,), jnp.int32),
        pltpu.VMEM((BPT, FACTOR), jnp.float32),
        pltpu.SemaphoreType.DMA,
    ],
)
def _gather_kernel(emb_hbm, ids0_hbm, ids1_hbm, ids2_hbm, out_hbm,
                   idx_v, rowbuf, sem):
    c = lax.axis_index("c")
    s = lax.axis_index("s")
    wid = s * NC + c
    base = wid * BPT
    for t, ids_hbm in enumerate((ids0_hbm, ids1_hbm, ids2_hbm)):
        pltpu.sync_copy(ids_hbm.at[pl.ds(base, BPT)], idx_v)
        pltpu.async_copy(emb_hbm.at[idx_v], rowbuf, sem).wait()
        pltpu.sync_copy(rowbuf, out_hbm.at[t, pl.ds(base, BPT)])


def _dinv(deg_ref):
    deg = jnp.sum(deg_ref[...], axis=0) + 1.0
    return lax.rsqrt(deg)[:, None]


def _b_body(x_ref, w_ref, deg_ref, y_ref):
    h = jnp.dot(x_ref[...], w_ref[...], preferred_element_type=jnp.float32)
    y_ref[...] = h * _dinv(deg_ref)


_tc_b = pl.pallas_call(
    _b_body, out_shape=jax.ShapeDtypeStruct((NPAD, HID_CH), jnp.float32)
)


def _d_body(p_ref, y1_ref, deg_ref, w2_ref, b1_ref, y2_ref):
    dinv = _dinv(deg_ref)
    z = p_ref[0] + p_ref[1] + y1_ref[...]
    h1 = jnp.maximum(z * dinv + b1_ref[...], 0.0)
    y2_ref[...] = jnp.dot(h1, w2_ref[...], preferred_element_type=jnp.float32) * dinv


_tc_d = pl.pallas_call(
    _d_body, out_shape=jax.ShapeDtypeStruct((NPAD, FACTOR), jnp.float32)
)


def _f_body(q_ref, y2_ref, deg_ref, b2_ref, emb_ref):
    z2 = q_ref[0] + q_ref[1] + y2_ref[...]
    emb_ref[...] = z2 * _dinv(deg_ref) + b2_ref[...]


_tc_f = pl.pallas_call(
    _f_body, out_shape=jax.ShapeDtypeStruct((NPAD, FACTOR), jnp.float32)
)


def _h_body(rows_ref, out_ref):
    a = rows_ref[0]
    pi = jnp.sum(a * rows_ref[1], axis=-1, keepdims=True)
    pj = jnp.sum(a * rows_ref[2], axis=-1, keepdims=True)
    out_ref[...] = jnp.concatenate([pi, pj], axis=1)


_tc_h = pl.pallas_call(
    _h_body, out_shape=jax.ShapeDtypeStruct((BATCH, 2), jnp.float32)
)


def kernel(drug_ids, disease_ids_i, disease_ids_j, x, edge_index, W1, b1, W2, b2):
    src = edge_index[0].astype(jnp.int32)
    dst = edge_index[1].astype(jnp.int32)
    pad = N_NODES + jnp.arange(EPAD - N_EDGES, dtype=jnp.int32) % (NPAD - N_NODES)
    srcs = jnp.concatenate([src, pad]).reshape(NW, NCHUNK, CH)
    dsts = jnp.concatenate([dst, pad]).reshape(NW, NCHUNK, CH)

    deg2 = _deg_kernel(dsts).reshape(NW, NPAD)

    x_p = jnp.concatenate([x, jnp.zeros((NPAD - N_NODES, IN_CH), x.dtype)])
    y1 = _tc_b(x_p, W1, deg2)

    zeros128 = jnp.zeros((NPAD, HID_CH), jnp.float32)
    p = _scatter128(y1, srcs, dsts, zeros128)
    y2 = _tc_d(p, y1, deg2, W2, b1)

    q = _scatter64(y2, srcs, dsts, jnp.zeros((NPAD, FACTOR), jnp.float32))
    emb_p = _tc_f(q, y2, deg2, b2)

    rows = _gather_kernel(
        emb_p,
        drug_ids.astype(jnp.int32),
        disease_ids_i.astype(jnp.int32),
        disease_ids_j.astype(jnp.int32),
    )
    preds = _tc_h(rows)
    return (preds[:, 0], preds[:, 1], emb_p[:N_NODES])


# speedup vs baseline: 34.7788x; 1.0073x over previous
import functools

import jax
import jax.numpy as jnp
import numpy as np
from jax import lax
from jax.experimental import pallas as pl
from jax.experimental.pallas import tpu as pltpu
from jax.experimental.pallas import tpu_sc as plsc

N_NODES = 10000
IN_CH = 128
HID_CH = 128
FACTOR = 64
N_EDGES = 320000
BATCH = 4096

NC = 2
NS = 16
NW = NC * NS
LANES = 16

CH = 64
NCHUNK = 160
NPHASE = 4
NCHP = NCHUNK // NPHASE
EPT = NCHUNK * CH
EPAD = NW * EPT
NPAD = 10112
RPT = NPAD // NS
BPT = BATCH // NW

def _mesh():
    return plsc.VectorSubcoreMesh(
        core_axis_name="c", subcore_axis_name="s", num_cores=NC, num_subcores=NS
    )


@functools.partial(
    pl.kernel,
    out_type=jax.ShapeDtypeStruct((NW * NPAD,), jnp.float32),
    mesh=_mesh(),
    scratch_types=[
        pltpu.VMEM((NCHUNK, CH), jnp.int32),
        pltpu.VMEM((NPAD,), jnp.float32),
    ],
    compiler_params=pltpu.CompilerParams(needs_layout_passes=False),
)
def _deg_kernel(dsts_hbm, out_hbm, dst_v, hist):
    c = lax.axis_index("c")
    s = lax.axis_index("s")
    wid = s * NC + c
    pltpu.sync_copy(dsts_hbm.at[wid], dst_v)

    def zbody(k, carry):
        hist[pl.ds(k * LANES, LANES)] = jnp.zeros((LANES,), jnp.float32)
        return carry

    lax.fori_loop(0, NPAD // LANES, zbody, 0)
    ones = jnp.ones((LANES,), jnp.float32)

    def body(j, carry):
        for k in range(CH // LANES):
            idx = dst_v[j, pl.ds(k * LANES, LANES)]
            plsc.addupdate_scatter(hist, [idx], ones)
        return carry

    lax.fori_loop(0, NCHUNK, body, 0)
    pltpu.sync_copy(hist, out_hbm.at[pl.ds(wid * NPAD, NPAD)])


NBUF = 4


def _make_scatter(width, tc_tiling=True):
    @functools.partial(
        pl.kernel,
        out_type=jax.ShapeDtypeStruct((NC, NPAD, width), jnp.float32),
        mesh=_mesh(),
        compiler_params=pltpu.CompilerParams(use_tc_tiling_on_sc=tc_tiling),
        scratch_types=(
            [
                pltpu.VMEM((NCHP, CH), jnp.int32),
                pltpu.VMEM((NCHP, CH), jnp.int32),
                pltpu.VMEM_SHARED((NPAD, width), jnp.float32),
            ]
            + [pltpu.VMEM((CH, width), jnp.float32)] * NBUF
            + [pltpu.SemaphoreType.DMA] * NBUF
        ),
    )
    def _scatter_kernel(y_hbm, srcs_hbm, dsts_hbm, zeros_hbm, out_hbm,
                        src_v, dst_v, acc, *bufs_sems):
        rowbufs = bufs_sems[:NBUF]
        sems = bufs_sems[NBUF:]
        c = lax.axis_index("c")
        s = lax.axis_index("s")
        wid = s * NC + c
        r0 = s * RPT
        pltpu.sync_copy(zeros_hbm, acc.at[pl.ds(r0, RPT)])
        plsc.subcore_barrier()

        for phase in range(NPHASE):
            pltpu.sync_copy(srcs_hbm.at[wid, pl.ds(phase * NCHP, NCHP)], src_v)
            pltpu.sync_copy(dsts_hbm.at[wid, pl.ds(phase * NCHP, NCHP)], dst_v)
            for b in range(NBUF):
                pltpu.async_copy(y_hbm.at[src_v.at[b]], rowbufs[b], sems[b])

            def body(g, carry):
                for b in range(NBUF):
                    j = g * NBUF + b
                    pltpu.make_async_copy(y_hbm.at[src_v.at[j]], rowbufs[b],
                                          sems[b]).wait()
                    pltpu.sync_copy(rowbufs[b], acc.at[dst_v.at[j]], add=True)
                    jn = j + NBUF

                    @pl.when(jn < NCHP)
                    def _():
                        pltpu.async_copy(y_hbm.at[src_v.at[jn]], rowbufs[b],
                                         sems[b])

                return carry

            lax.fori_loop(0, NCHP // NBUF, body, 0)

        plsc.subcore_barrier()
        pltpu.sync_copy(acc.at[pl.ds(r0, RPT)], out_hbm.at[c, pl.ds(r0, RPT)])

    return _scatter_kernel


_scatter128 = _make_scatter(HID_CH)
_scatter64 = _make_scatter(FACTOR, tc_tiling=False)


@functools.partial(
    pl.kernel,
    out_type=jax.ShapeDtypeStruct((3, BATCH, FACTOR), jnp.float32),
    mesh=_mesh(),
    compiler_params=pltpu.CompilerParams(use_tc_tiling_on_sc=False),
    scratch_types=[
        pltpu.VMEM((BPT,), jnp.int32),
        pltpu.VMEM((BPT, FACTOR), jnp.float32),
        pltpu.SemaphoreType.DMA,
    ],
)
def _gather_kernel(emb_hbm, ids0_hbm, ids1_hbm, ids2_hbm, out_hbm,
                   idx_v, rowbuf, sem):
    c = lax.axis_index("c")
    s = lax.axis_index("s")
    wid = s * NC + c
    base = wid * BPT
    for t, ids_hbm in enumerate((ids0_hbm, ids1_hbm, ids2_hbm)):
        pltpu.sync_copy(ids_hbm.at[pl.ds(base, BPT)], idx_v)
        pltpu.async_copy(emb_hbm.at[idx_v], rowbuf, sem).wait()
        pltpu.sync_copy(rowbuf, out_hbm.at[t, pl.ds(base, BPT)])


def _dinv(deg_ref):
    total = deg_ref[pl.ds(0, NPAD)]
    for w in range(1, NW):
        total = total + deg_ref[pl.ds(w * NPAD, NPAD)]
    return lax.rsqrt(total + 1.0)[:, None]


def _b_body(x_ref, w_ref, deg_ref, y_ref):
    h = jnp.dot(x_ref[...], w_ref[...], preferred_element_type=jnp.float32)
    y_ref[pl.ds(0, N_NODES)] = h * _dinv(deg_ref)[:N_NODES]
    y_ref[pl.ds(N_NODES, NPAD - N_NODES)] = jnp.zeros(
        (NPAD - N_NODES, HID_CH), jnp.float32
    )


_tc_b = pl.pallas_call(
    _b_body, out_shape=jax.ShapeDtypeStruct((NPAD, HID_CH), jnp.float32)
)


def _d_body(p_ref, y1_ref, deg_ref, w2_ref, b1_ref, y2_ref):
    dinv = _dinv(deg_ref)
    z = p_ref[0] + p_ref[1] + y1_ref[...]
    h1 = jnp.maximum(z * dinv + b1_ref[...], 0.0)
    y2_ref[...] = jnp.dot(h1, w2_ref[...], preferred_element_type=jnp.float32) * dinv


_tc_d = pl.pallas_call(
    _d_body, out_shape=jax.ShapeDtypeStruct((NPAD, FACTOR), jnp.float32)
)


def _f_body(q_ref, y2_ref, deg_ref, b2_ref, emb_ref, node_ref):
    z2 = q_ref[0] + q_ref[1] + y2_ref[...]
    e = z2 * _dinv(deg_ref) + b2_ref[...]
    emb_ref[...] = e
    node_ref[...] = e[:N_NODES]


_tc_f = pl.pallas_call(
    _f_body,
    out_shape=[
        jax.ShapeDtypeStruct((NPAD, FACTOR), jnp.float32),
        jax.ShapeDtypeStruct((N_NODES, FACTOR), jnp.float32),
    ],
)


def _h_body(rows_ref, out_ref):
    a = rows_ref[0]
    pi = jnp.sum(a * rows_ref[1], axis=-1, keepdims=True)
    pj = jnp.sum(a * rows_ref[2], axis=-1, keepdims=True)
    out_ref[...] = jnp.concatenate([pi, pj], axis=1)


_tc_h = pl.pallas_call(
    _h_body, out_shape=jax.ShapeDtypeStruct((BATCH, 2), jnp.float32)
)


_PAD_IDX = np.asarray(
    N_NODES + np.arange(EPAD - N_EDGES) % (NPAD - N_NODES), dtype=np.int32
)


def kernel(drug_ids, disease_ids_i, disease_ids_j, x, edge_index, W1, b1, W2, b2):
    src = edge_index[0].astype(jnp.int32)
    dst = edge_index[1].astype(jnp.int32)
    pad = jnp.asarray(_PAD_IDX)
    srcs = jnp.concatenate([src, pad]).reshape(NW, NCHUNK, CH)
    dsts = jnp.concatenate([dst, pad]).reshape(NW, NCHUNK, CH)

    degp = _deg_kernel(dsts)

    y1 = _tc_b(x, W1, degp)

    p = _scatter128(y1, srcs, dsts, jnp.zeros((RPT, HID_CH), jnp.float32))
    y2 = _tc_d(p, y1, degp, W2, b1)

    q = _scatter64(y2, srcs, dsts, jnp.zeros((RPT, FACTOR), jnp.float32))
    emb_p, node_emb = _tc_f(q, y2, degp, b2)

    rows = _gather_kernel(
        emb_p,
        drug_ids.astype(jnp.int32),
        disease_ids_i.astype(jnp.int32),
        disease_ids_j.astype(jnp.int32),
    )
    preds = _tc_h(rows)
    return (preds[:, 0], preds[:, 1], node_emb)


# speedup vs baseline: 36.4581x; 1.0483x over previous
import functools

import jax
import jax.numpy as jnp
import numpy as np
from jax import lax
from jax.experimental import pallas as pl
from jax.experimental.pallas import tpu as pltpu
from jax.experimental.pallas import tpu_sc as plsc

N_NODES = 10000
IN_CH = 128
HID_CH = 128
FACTOR = 64
N_EDGES = 320000
BATCH = 4096

NC = 2
NS = 16
NW = NC * NS
LANES = 16

CH = 64
NCHUNK = 160
NPHASE = 4
NCHP = NCHUNK // NPHASE
EPT = NCHUNK * CH
EPAD = NW * EPT
NPAD = 10112
RPT = NPAD // NS
BPT = BATCH // NW

def _mesh():
    return plsc.VectorSubcoreMesh(
        core_axis_name="c", subcore_axis_name="s", num_cores=NC, num_subcores=NS
    )


@functools.partial(
    pl.kernel,
    out_type=jax.ShapeDtypeStruct((NW * NPAD,), jnp.float32),
    mesh=_mesh(),
    scratch_types=[
        pltpu.VMEM((NCHUNK, CH), jnp.int32),
        pltpu.VMEM((NPAD,), jnp.float32),
    ],
    compiler_params=pltpu.CompilerParams(needs_layout_passes=False),
)
def _deg_kernel(dsts_hbm, out_hbm, dst_v, hist):
    c = lax.axis_index("c")
    s = lax.axis_index("s")
    wid = s * NC + c
    pltpu.sync_copy(dsts_hbm.at[wid], dst_v)

    def zbody(k, carry):
        hist[pl.ds(k * LANES, LANES)] = jnp.zeros((LANES,), jnp.float32)
        return carry

    lax.fori_loop(0, NPAD // LANES, zbody, 0)
    ones = jnp.ones((LANES,), jnp.float32)

    def body(j, carry):
        for k in range(CH // LANES):
            idx = dst_v[j, pl.ds(k * LANES, LANES)]
            plsc.addupdate_scatter(hist, [idx], ones)
        return carry

    lax.fori_loop(0, NCHUNK, body, 0)
    pltpu.sync_copy(hist, out_hbm.at[pl.ds(wid * NPAD, NPAD)])


NBUF = 4


def _make_scatter(width, tc_tiling=True):
    @functools.partial(
        pl.kernel,
        out_type=jax.ShapeDtypeStruct((NC, NPAD, width), jnp.float32),
        mesh=_mesh(),
        compiler_params=pltpu.CompilerParams(use_tc_tiling_on_sc=tc_tiling),
        scratch_types=(
            [
                pltpu.VMEM((NCHP, CH), jnp.int32),
                pltpu.VMEM((NCHP, CH), jnp.int32),
                pltpu.VMEM_SHARED((NPAD, width), jnp.float32),
            ]
            + [pltpu.VMEM((CH, width), jnp.float32)] * NBUF
            + [pltpu.SemaphoreType.DMA] * NBUF
        ),
    )
    def _scatter_kernel(y_hbm, srcs_hbm, dsts_hbm, zeros_hbm, out_hbm,
                        src_v, dst_v, acc, *bufs_sems):
        rowbufs = bufs_sems[:NBUF]
        sems = bufs_sems[NBUF:]
        c = lax.axis_index("c")
        s = lax.axis_index("s")
        wid = s * NC + c
        r0 = s * RPT
        pltpu.sync_copy(zeros_hbm, acc.at[pl.ds(r0, RPT)])
        plsc.subcore_barrier()

        for phase in range(NPHASE):
            pltpu.sync_copy(srcs_hbm.at[wid, pl.ds(phase * NCHP, NCHP)], src_v)
            pltpu.sync_copy(dsts_hbm.at[wid, pl.ds(phase * NCHP, NCHP)], dst_v)
            for b in range(NBUF):
                pltpu.async_copy(y_hbm.at[src_v.at[b]], rowbufs[b], sems[b])

            def body(g, carry):
                for b in range(NBUF):
                    j = g * NBUF + b
                    pltpu.make_async_copy(y_hbm.at[src_v.at[j]], rowbufs[b],
                                          sems[b]).wait()
                    pltpu.sync_copy(rowbufs[b], acc.at[dst_v.at[j]], add=True)
                    jn = j + NBUF

                    @pl.when(jn < NCHP)
                    def _():
                        pltpu.async_copy(y_hbm.at[src_v.at[jn]], rowbufs[b],
                                         sems[b])

                return carry

            lax.fori_loop(0, NCHP // NBUF, body, 0)

        plsc.subcore_barrier()
        pltpu.sync_copy(acc.at[pl.ds(r0, RPT)], out_hbm.at[c, pl.ds(r0, RPT)])

    return _scatter_kernel


_scatter128 = _make_scatter(HID_CH)
_scatter64 = _make_scatter(FACTOR, tc_tiling=False)


@functools.partial(
    pl.kernel,
    out_type=jax.ShapeDtypeStruct((2, BATCH), jnp.float32),
    mesh=_mesh(),
    compiler_params=pltpu.CompilerParams(
        use_tc_tiling_on_sc=False, needs_layout_passes=False
    ),
    scratch_types=[
        pltpu.VMEM((BPT,), jnp.int32),
        pltpu.VMEM((BPT, FACTOR), jnp.float32),
        pltpu.VMEM((BPT, FACTOR), jnp.float32),
        pltpu.VMEM((BPT, FACTOR), jnp.float32),
        pltpu.VMEM((BPT,), jnp.float32),
        pltpu.VMEM((BPT,), jnp.float32),
        pltpu.SemaphoreType.DMA,
    ],
)
def _gather_kernel(emb_hbm, ids0_hbm, ids1_hbm, ids2_hbm, out_hbm,
                   idx_v, bufa, bufi, bufj, predi, predj, sem):
    c = lax.axis_index("c")
    s = lax.axis_index("s")
    wid = s * NC + c
    base = wid * BPT
    for ids_hbm, buf in ((ids0_hbm, bufa), (ids1_hbm, bufi), (ids2_hbm, bufj)):
        pltpu.sync_copy(ids_hbm.at[pl.ds(base, BPT)], idx_v)
        pltpu.async_copy(emb_hbm.at[idx_v], buf, sem).wait()

    lane0 = lax.iota(jnp.int32, LANES) == 0

    def body(r, carry):
        acc_i = jnp.zeros((LANES,), jnp.float32)
        acc_j = jnp.zeros((LANES,), jnp.float32)
        for cb in range(FACTOR // LANES):
            va = bufa[r, pl.ds(cb * LANES, LANES)]
            acc_i = acc_i + va * bufi[r, pl.ds(cb * LANES, LANES)]
            acc_j = acc_j + va * bufj[r, pl.ds(cb * LANES, LANES)]
        ridx = jnp.full((LANES,), r, jnp.int32)
        plsc.store_scatter(predi, [ridx], jnp.full((LANES,), jnp.sum(acc_i)),
                           mask=lane0)
        plsc.store_scatter(predj, [ridx], jnp.full((LANES,), jnp.sum(acc_j)),
                           mask=lane0)
        return carry

    lax.fori_loop(0, BPT, body, 0)
    pltpu.sync_copy(predi, out_hbm.at[0, pl.ds(base, BPT)])
    pltpu.sync_copy(predj, out_hbm.at[1, pl.ds(base, BPT)])


def _dinv(deg_ref):
    total = deg_ref[pl.ds(0, NPAD)]
    for w in range(1, NW):
        total = total + deg_ref[pl.ds(w * NPAD, NPAD)]
    return lax.rsqrt(total + 1.0)[:, None]


def _b_body(x_ref, w_ref, deg_ref, y_ref):
    h = jnp.dot(x_ref[...], w_ref[...], preferred_element_type=jnp.float32)
    y_ref[pl.ds(0, N_NODES)] = h * _dinv(deg_ref)[:N_NODES]
    y_ref[pl.ds(N_NODES, NPAD - N_NODES)] = jnp.zeros(
        (NPAD - N_NODES, HID_CH), jnp.float32
    )


_tc_b = pl.pallas_call(
    _b_body, out_shape=jax.ShapeDtypeStruct((NPAD, HID_CH), jnp.float32)
)


def _d_body(p_ref, y1_ref, deg_ref, w2_ref, b1_ref, y2_ref):
    dinv = _dinv(deg_ref)
    z = p_ref[0] + p_ref[1] + y1_ref[...]
    h1 = jnp.maximum(z * dinv + b1_ref[...], 0.0)
    y2_ref[...] = jnp.dot(h1, w2_ref[...], preferred_element_type=jnp.float32) * dinv


_tc_d = pl.pallas_call(
    _d_body, out_shape=jax.ShapeDtypeStruct((NPAD, FACTOR), jnp.float32)
)


def _f_body(q_ref, y2_ref, deg_ref, b2_ref, emb_ref, node_ref):
    z2 = q_ref[0] + q_ref[1] + y2_ref[...]
    e = z2 * _dinv(deg_ref) + b2_ref[...]
    emb_ref[...] = e
    node_ref[...] = e[:N_NODES]


_tc_f = pl.pallas_call(
    _f_body,
    out_shape=[
        jax.ShapeDtypeStruct((NPAD, FACTOR), jnp.float32),
        jax.ShapeDtypeStruct((N_NODES, FACTOR), jnp.float32),
    ],
)


_PAD_IDX = np.asarray(
    N_NODES + np.arange(EPAD - N_EDGES) % (NPAD - N_NODES), dtype=np.int32
)


def kernel(drug_ids, disease_ids_i, disease_ids_j, x, edge_index, W1, b1, W2, b2):
    src = edge_index[0].astype(jnp.int32)
    dst = edge_index[1].astype(jnp.int32)
    pad = jnp.asarray(_PAD_IDX)
    srcs = jnp.concatenate([src, pad]).reshape(NW, NCHUNK, CH)
    dsts = jnp.concatenate([dst, pad]).reshape(NW, NCHUNK, CH)

    degp = _deg_kernel(dsts)

    y1 = _tc_b(x, W1, degp)

    p = _scatter128(y1, srcs, dsts, jnp.zeros((RPT, HID_CH), jnp.float32))
    y2 = _tc_d(p, y1, degp, W2, b1)

    q = _scatter64(y2, srcs, dsts, jnp.zeros((RPT, FACTOR), jnp.float32))
    emb_p, node_emb = _tc_f(q, y2, degp, b2)

    preds = _gather_kernel(
        emb_p,
        drug_ids.astype(jnp.int32),
        disease_ids_i.astype(jnp.int32),
        disease_ids_j.astype(jnp.int32),
    )
    return (preds[0], preds[1], node_emb)


# speedup vs baseline: 36.7996x; 1.0094x over previous
import functools

import jax
import jax.numpy as jnp
import numpy as np
from jax import lax
from jax.experimental import pallas as pl
from jax.experimental.pallas import tpu as pltpu
from jax.experimental.pallas import tpu_sc as plsc

N_NODES = 10000
IN_CH = 128
HID_CH = 128
FACTOR = 64
N_EDGES = 320000
BATCH = 4096

NC = 2
NS = 16
NW = NC * NS
LANES = 16

CH = 64
NCHUNK = 160
NPHASE = 4
NCHP = NCHUNK // NPHASE
EPT = NCHUNK * CH
EPAD = NW * EPT
NPAD = 10112
RPT = NPAD // NS
BPT = BATCH // NW

def _mesh():
    return plsc.VectorSubcoreMesh(
        core_axis_name="c", subcore_axis_name="s", num_cores=NC, num_subcores=NS
    )


@functools.partial(
    pl.kernel,
    out_type=jax.ShapeDtypeStruct((NW * NPAD,), jnp.float32),
    mesh=_mesh(),
    scratch_types=[
        pltpu.VMEM((NCHUNK, CH), jnp.int32),
        pltpu.VMEM((NPAD,), jnp.float32),
    ],
    compiler_params=pltpu.CompilerParams(
        needs_layout_passes=False, use_tc_tiling_on_sc=False
    ),
)
def _deg_kernel(dsts_hbm, out_hbm, dst_v, hist):
    c = lax.axis_index("c")
    s = lax.axis_index("s")
    wid = s * NC + c
    pltpu.sync_copy(dsts_hbm.at[wid], dst_v)

    def zbody(k, carry):
        hist[pl.ds(k * LANES, LANES)] = jnp.zeros((LANES,), jnp.float32)
        return carry

    lax.fori_loop(0, NPAD // LANES, zbody, 0)
    ones = jnp.ones((LANES,), jnp.float32)

    def body(j, carry):
        for k in range(CH // LANES):
            idx = dst_v[j, pl.ds(k * LANES, LANES)]
            plsc.addupdate_scatter(hist, [idx], ones)
        return carry

    lax.fori_loop(0, NCHUNK, body, 0)
    pltpu.sync_copy(hist, out_hbm.at[pl.ds(wid * NPAD, NPAD)])


NBUF = 4


def _make_scatter(width, tc_tiling=True):
    @functools.partial(
        pl.kernel,
        out_type=jax.ShapeDtypeStruct((NC, NPAD, width), jnp.float32),
        mesh=_mesh(),
        compiler_params=pltpu.CompilerParams(use_tc_tiling_on_sc=tc_tiling),
        scratch_types=(
            [
                pltpu.VMEM((NCHP, CH), jnp.int32),
                pltpu.VMEM((NCHP, CH), jnp.int32),
                pltpu.VMEM_SHARED((NPAD, width), jnp.float32),
            ]
            + [pltpu.VMEM((CH, width), jnp.float32)] * NBUF
            + [pltpu.SemaphoreType.DMA] * NBUF
        ),
    )
    def _scatter_kernel(y_hbm, srcs_hbm, dsts_hbm, zeros_hbm, out_hbm,
                        src_v, dst_v, acc, *bufs_sems):
        rowbufs = bufs_sems[:NBUF]
        sems = bufs_sems[NBUF:]
        c = lax.axis_index("c")
        s = lax.axis_index("s")
        wid = s * NC + c
        r0 = s * RPT
        pltpu.sync_copy(zeros_hbm, acc.at[pl.ds(r0, RPT)])
        plsc.subcore_barrier()

        for phase in range(NPHASE):
            pltpu.sync_copy(srcs_hbm.at[wid, pl.ds(phase * NCHP, NCHP)], src_v)
            pltpu.sync_copy(dsts_hbm.at[wid, pl.ds(phase * NCHP, NCHP)], dst_v)
            for b in range(NBUF):
                pltpu.async_copy(y_hbm.at[src_v.at[b]], rowbufs[b], sems[b])

            def body(g, carry):
                for b in range(NBUF):
                    j = g * NBUF + b
                    pltpu.make_async_copy(y_hbm.at[src_v.at[j]], rowbufs[b],
                                          sems[b]).wait()
                    pltpu.sync_copy(rowbufs[b], acc.at[dst_v.at[j]], add=True)
                    jn = j + NBUF

                    @pl.when(jn < NCHP)
                    def _():
                        pltpu.async_copy(y_hbm.at[src_v.at[jn]], rowbufs[b],
                                         sems[b])

                return carry

            lax.fori_loop(0, NCHP // NBUF, body, 0)

        plsc.subcore_barrier()
        pltpu.sync_copy(acc.at[pl.ds(r0, RPT)], out_hbm.at[c, pl.ds(r0, RPT)])

    return _scatter_kernel


_scatter128 = _make_scatter(HID_CH, tc_tiling=False)
_scatter64 = _make_scatter(FACTOR, tc_tiling=False)


@functools.partial(
    pl.kernel,
    out_type=jax.ShapeDtypeStruct((2, BATCH), jnp.float32),
    mesh=_mesh(),
    compiler_params=pltpu.CompilerParams(
        use_tc_tiling_on_sc=False, needs_layout_passes=False
    ),
    scratch_types=[
        pltpu.VMEM((BPT,), jnp.int32),
        pltpu.VMEM((BPT, FACTOR), jnp.float32),
        pltpu.VMEM((BPT, FACTOR), jnp.float32),
        pltpu.VMEM((BPT, FACTOR), jnp.float32),
        pltpu.VMEM((BPT,), jnp.float32),
        pltpu.VMEM((BPT,), jnp.float32),
        pltpu.SemaphoreType.DMA,
    ],
)
def _gather_kernel(emb_hbm, ids0_hbm, ids1_hbm, ids2_hbm, out_hbm,
                   idx_v, bufa, bufi, bufj, predi, predj, sem):
    c = lax.axis_index("c")
    s = lax.axis_index("s")
    wid = s * NC + c
    base = wid * BPT
    for ids_hbm, buf in ((ids0_hbm, bufa), (ids1_hbm, bufi), (ids2_hbm, bufj)):
        pltpu.sync_copy(ids_hbm.at[pl.ds(base, BPT)], idx_v)
        pltpu.async_copy(emb_hbm.at[idx_v], buf, sem).wait()

    lane0 = lax.iota(jnp.int32, LANES) == 0

    def body(r, carry):
        acc_i = jnp.zeros((LANES,), jnp.float32)
        acc_j = jnp.zeros((LANES,), jnp.float32)
        for cb in range(FACTOR // LANES):
            va = bufa[r, pl.ds(cb * LANES, LANES)]
            acc_i = acc_i + va * bufi[r, pl.ds(cb * LANES, LANES)]
            acc_j = acc_j + va * bufj[r, pl.ds(cb * LANES, LANES)]
        ridx = jnp.full((LANES,), r, jnp.int32)
        plsc.store_scatter(predi, [ridx], jnp.full((LANES,), jnp.sum(acc_i)),
                           mask=lane0)
        plsc.store_scatter(predj, [ridx], jnp.full((LANES,), jnp.sum(acc_j)),
                           mask=lane0)
        return carry

    lax.fori_loop(0, BPT, body, 0)
    pltpu.sync_copy(predi, out_hbm.at[0, pl.ds(base, BPT)])
    pltpu.sync_copy(predj, out_hbm.at[1, pl.ds(base, BPT)])


def _dinv(deg_ref):
    total = deg_ref[pl.ds(0, NPAD)]
    for w in range(1, NW):
        total = total + deg_ref[pl.ds(w * NPAD, NPAD)]
    return lax.rsqrt(total + 1.0)[:, None]


def _b_body(x_ref, w_ref, deg_ref, y_ref):
    h = jnp.dot(x_ref[...], w_ref[...], preferred_element_type=jnp.float32)
    y_ref[pl.ds(0, N_NODES)] = h * _dinv(deg_ref)[:N_NODES]
    y_ref[pl.ds(N_NODES, NPAD - N_NODES)] = jnp.zeros(
        (NPAD - N_NODES, HID_CH), jnp.float32
    )


_tc_b = pl.pallas_call(
    _b_body, out_shape=jax.ShapeDtypeStruct((NPAD, HID_CH), jnp.float32)
)


def _d_body(p_ref, y1_ref, deg_ref, w2_ref, b1_ref, y2_ref):
    dinv = _dinv(deg_ref)
    z = p_ref[0] + p_ref[1] + y1_ref[...]
    h1 = jnp.maximum(z * dinv + b1_ref[...], 0.0)
    y2_ref[...] = jnp.dot(h1, w2_ref[...], preferred_element_type=jnp.float32) * dinv


_tc_d = pl.pallas_call(
    _d_body, out_shape=jax.ShapeDtypeStruct((NPAD, FACTOR), jnp.float32)
)


def _f_body(q_ref, y2_ref, deg_ref, b2_ref, emb_ref, node_ref):
    z2 = q_ref[0] + q_ref[1] + y2_ref[...]
    e = z2 * _dinv(deg_ref) + b2_ref[...]
    emb_ref[...] = e
    node_ref[...] = e[:N_NODES]


_tc_f = pl.pallas_call(
    _f_body,
    out_shape=[
        jax.ShapeDtypeStruct((NPAD, FACTOR), jnp.float32),
        jax.ShapeDtypeStruct((N_NODES, FACTOR), jnp.float32),
    ],
)


_PAD_IDX = np.asarray(
    N_NODES + np.arange(EPAD - N_EDGES) % (NPAD - N_NODES), dtype=np.int32
)


def kernel(drug_ids, disease_ids_i, disease_ids_j, x, edge_index, W1, b1, W2, b2):
    src = edge_index[0].astype(jnp.int32)
    dst = edge_index[1].astype(jnp.int32)
    pad = jnp.asarray(_PAD_IDX)
    srcs = jnp.concatenate([src, pad]).reshape(NW, NCHUNK, CH)
    dsts = jnp.concatenate([dst, pad]).reshape(NW, NCHUNK, CH)

    degp = _deg_kernel(dsts)

    y1 = _tc_b(x, W1, degp)

    p = _scatter128(y1, srcs, dsts, jnp.zeros((RPT, HID_CH), jnp.float32))
    y2 = _tc_d(p, y1, degp, W2, b1)

    q = _scatter64(y2, srcs, dsts, jnp.zeros((RPT, FACTOR), jnp.float32))
    emb_p, node_emb = _tc_f(q, y2, degp, b2)

    preds = _gather_kernel(
        emb_p,
        drug_ids.astype(jnp.int32),
        disease_ids_i.astype(jnp.int32),
        disease_ids_j.astype(jnp.int32),
    )
    return (preds[0], preds[1], node_emb)


# speedup vs baseline: 36.8245x; 1.0007x over previous
import functools

import jax
import jax.numpy as jnp
import numpy as np
from jax import lax
from jax.experimental import pallas as pl
from jax.experimental.pallas import tpu as pltpu
from jax.experimental.pallas import tpu_sc as plsc

N_NODES = 10000
IN_CH = 128
HID_CH = 128
FACTOR = 64
N_EDGES = 320000
BATCH = 4096

NC = 2
NS = 16
NW = NC * NS
LANES = 16

CH = 64
NCHUNK = 160
NPHASE = 4
NCHP = NCHUNK // NPHASE
EPT = NCHUNK * CH
EPAD = NW * EPT
NPAD = 10112
RPT = NPAD // NS
BPT = BATCH // NW

def _mesh():
    return plsc.VectorSubcoreMesh(
        core_axis_name="c", subcore_axis_name="s", num_cores=NC, num_subcores=NS
    )


@functools.partial(
    pl.kernel,
    out_type=jax.ShapeDtypeStruct((NW * NPAD,), jnp.float32),
    mesh=_mesh(),
    scratch_types=[
        pltpu.VMEM((EPT,), jnp.int32),
        pltpu.VMEM((NPAD,), jnp.float32),
    ],
    compiler_params=pltpu.CompilerParams(
        needs_layout_passes=False, use_tc_tiling_on_sc=False
    ),
)
def _deg_kernel(dsts_hbm, out_hbm, dst_v, hist):
    c = lax.axis_index("c")
    s = lax.axis_index("s")
    wid = s * NC + c
    pltpu.sync_copy(dsts_hbm.at[pl.ds(wid * EPT, EPT)], dst_v)

    def zbody(k, carry):
        hist[pl.ds(k * LANES, LANES)] = jnp.zeros((LANES,), jnp.float32)
        return carry

    lax.fori_loop(0, NPAD // LANES, zbody, 0)
    ones = jnp.ones((LANES,), jnp.float32)

    def body(j, carry):
        for k in range(4):
            idx = dst_v[pl.ds((j * 4 + k) * LANES, LANES)]
            plsc.addupdate_scatter(hist, [idx], ones)
        return carry

    lax.fori_loop(0, EPT // (4 * LANES), body, 0)
    pltpu.sync_copy(hist, out_hbm.at[pl.ds(wid * NPAD, NPAD)])


NBUF = 4


def _make_scatter(width, tc_tiling=True):
    @functools.partial(
        pl.kernel,
        out_type=jax.ShapeDtypeStruct((NC, NPAD, width), jnp.float32),
        mesh=_mesh(),
        compiler_params=pltpu.CompilerParams(use_tc_tiling_on_sc=tc_tiling),
        scratch_types=(
            [
                pltpu.VMEM((NCHP * CH,), jnp.int32),
                pltpu.VMEM((NCHP * CH,), jnp.int32),
                pltpu.VMEM_SHARED((NPAD, width), jnp.float32),
            ]
            + [pltpu.VMEM((CH, width), jnp.float32)] * NBUF
            + [pltpu.SemaphoreType.DMA] * NBUF
        ),
    )
    def _scatter_kernel(y_hbm, srcs_hbm, dsts_hbm, zeros_hbm, out_hbm,
                        src_v, dst_v, acc, *bufs_sems):
        rowbufs = bufs_sems[:NBUF]
        sems = bufs_sems[NBUF:]
        c = lax.axis_index("c")
        s = lax.axis_index("s")
        wid = s * NC + c
        r0 = s * RPT
        pltpu.sync_copy(zeros_hbm, acc.at[pl.ds(r0, RPT)])
        plsc.subcore_barrier()

        for phase in range(NPHASE):
            e0 = wid * EPT + phase * (NCHP * CH)
            pltpu.sync_copy(srcs_hbm.at[pl.ds(e0, NCHP * CH)], src_v)
            pltpu.sync_copy(dsts_hbm.at[pl.ds(e0, NCHP * CH)], dst_v)
            for b in range(NBUF):
                pltpu.async_copy(y_hbm.at[src_v.at[pl.ds(b * CH, CH)]],
                                 rowbufs[b], sems[b])

            def body(g, carry):
                for b in range(NBUF):
                    j = g * NBUF + b
                    pltpu.make_async_copy(
                        y_hbm.at[src_v.at[pl.ds(j * CH, CH)]], rowbufs[b],
                        sems[b]).wait()
                    pltpu.sync_copy(rowbufs[b],
                                    acc.at[dst_v.at[pl.ds(j * CH, CH)]],
                                    add=True)
                    jn = j + NBUF

                    @pl.when(jn < NCHP)
                    def _():
                        pltpu.async_copy(
                            y_hbm.at[src_v.at[pl.ds(jn * CH, CH)]],
                            rowbufs[b], sems[b])

                return carry

            lax.fori_loop(0, NCHP // NBUF, body, 0)

        plsc.subcore_barrier()
        pltpu.sync_copy(acc.at[pl.ds(r0, RPT)], out_hbm.at[c, pl.ds(r0, RPT)])

    return _scatter_kernel


_scatter128 = _make_scatter(HID_CH, tc_tiling=False)
_scatter64 = _make_scatter(FACTOR, tc_tiling=False)


@functools.partial(
    pl.kernel,
    out_type=jax.ShapeDtypeStruct((2, BATCH), jnp.float32),
    mesh=_mesh(),
    compiler_params=pltpu.CompilerParams(
        use_tc_tiling_on_sc=False, needs_layout_passes=False
    ),
    scratch_types=[
        pltpu.VMEM((BPT,), jnp.int32),
        pltpu.VMEM((BPT, FACTOR), jnp.float32),
        pltpu.VMEM((BPT, FACTOR), jnp.float32),
        pltpu.VMEM((BPT, FACTOR), jnp.float32),
        pltpu.VMEM((BPT,), jnp.float32),
        pltpu.VMEM((BPT,), jnp.float32),
        pltpu.SemaphoreType.DMA,
    ],
)
def _gather_kernel(emb_hbm, ids0_hbm, ids1_hbm, ids2_hbm, out_hbm,
                   idx_v, bufa, bufi, bufj, predi, predj, sem):
    c = lax.axis_index("c")
    s = lax.axis_index("s")
    wid = s * NC + c
    base = wid * BPT
    for ids_hbm, buf in ((ids0_hbm, bufa), (ids1_hbm, bufi), (ids2_hbm, bufj)):
        pltpu.sync_copy(ids_hbm.at[pl.ds(base, BPT)], idx_v)
        pltpu.async_copy(emb_hbm.at[idx_v], buf, sem).wait()

    lane0 = lax.iota(jnp.int32, LANES) == 0

    def body(r, carry):
        acc_i = jnp.zeros((LANES,), jnp.float32)
        acc_j = jnp.zeros((LANES,), jnp.float32)
        for cb in range(FACTOR // LANES):
            va = bufa[r, pl.ds(cb * LANES, LANES)]
            acc_i = acc_i + va * bufi[r, pl.ds(cb * LANES, LANES)]
            acc_j = acc_j + va * bufj[r, pl.ds(cb * LANES, LANES)]
        ridx = jnp.full((LANES,), r, jnp.int32)
        plsc.store_scatter(predi, [ridx], jnp.full((LANES,), jnp.sum(acc_i)),
                           mask=lane0)
        plsc.store_scatter(predj, [ridx], jnp.full((LANES,), jnp.sum(acc_j)),
                           mask=lane0)
        return carry

    lax.fori_loop(0, BPT, body, 0)
    pltpu.sync_copy(predi, out_hbm.at[0, pl.ds(base, BPT)])
    pltpu.sync_copy(predj, out_hbm.at[1, pl.ds(base, BPT)])


def _dinv(deg_ref):
    total = deg_ref[pl.ds(0, NPAD)]
    for w in range(1, NW):
        total = total + deg_ref[pl.ds(w * NPAD, NPAD)]
    return lax.rsqrt(total + 1.0)[:, None]


def _b_body(x_ref, w_ref, deg_ref, y_ref):
    h = jnp.dot(x_ref[...], w_ref[...], preferred_element_type=jnp.float32)
    y_ref[pl.ds(0, N_NODES)] = h * _dinv(deg_ref)[:N_NODES]
    y_ref[pl.ds(N_NODES, NPAD - N_NODES)] = jnp.zeros(
        (NPAD - N_NODES, HID_CH), jnp.float32
    )


_tc_b = pl.pallas_call(
    _b_body, out_shape=jax.ShapeDtypeStruct((NPAD, HID_CH), jnp.float32)
)


def _d_body(p_ref, y1_ref, deg_ref, w2_ref, b1_ref, y2_ref):
    dinv = _dinv(deg_ref)
    z = p_ref[0] + p_ref[1] + y1_ref[...]
    h1 = jnp.maximum(z * dinv + b1_ref[...], 0.0)
    y2_ref[...] = jnp.dot(h1, w2_ref[...], preferred_element_type=jnp.float32) * dinv


_tc_d = pl.pallas_call(
    _d_body, out_shape=jax.ShapeDtypeStruct((NPAD, FACTOR), jnp.float32)
)


def _f_body(q_ref, y2_ref, deg_ref, b2_ref, emb_ref, node_ref):
    z2 = q_ref[0] + q_ref[1] + y2_ref[...]
    e = z2 * _dinv(deg_ref) + b2_ref[...]
    emb_ref[...] = e
    node_ref[...] = e[:N_NODES]


_tc_f = pl.pallas_call(
    _f_body,
    out_shape=[
        jax.ShapeDtypeStruct((NPAD, FACTOR), jnp.float32),
        jax.ShapeDtypeStruct((N_NODES, FACTOR), jnp.float32),
    ],
)


_PAD_IDX = np.asarray(
    N_NODES + np.arange(EPAD - N_EDGES) % (NPAD - N_NODES), dtype=np.int32
)


def kernel(drug_ids, disease_ids_i, disease_ids_j, x, edge_index, W1, b1, W2, b2):
    src = edge_index[0].astype(jnp.int32)
    dst = edge_index[1].astype(jnp.int32)
    pad = jnp.asarray(_PAD_IDX)
    srcs = jnp.concatenate([src, pad])
    dsts = jnp.concatenate([dst, pad])

    degp = _deg_kernel(dsts)

    y1 = _tc_b(x, W1, degp)

    p = _scatter128(y1, srcs, dsts, jnp.zeros((RPT, HID_CH), jnp.float32))
    y2 = _tc_d(p, y1, degp, W2, b1)

    q = _scatter64(y2, srcs, dsts, jnp.zeros((RPT, FACTOR), jnp.float32))
    emb_p, node_emb = _tc_f(q, y2, degp, b2)

    preds = _gather_kernel(
        emb_p,
        drug_ids.astype(jnp.int32),
        disease_ids_i.astype(jnp.int32),
        disease_ids_j.astype(jnp.int32),
    )
    return (preds[0], preds[1], node_emb)
